# initial kernel scaffold (unmeasured)
import jax
import jax.numpy as jnp
from jax import lax
from jax.experimental import pallas as pl
from jax.experimental.pallas import tpu as pltpu

N_DEV = 4
M = 4096
N_OUT = 2048
CHUNK = M // N_DEV


def _gelu(y):
    c = 0.7978845608028654
    return 0.5 * y * (1.0 + jnp.tanh(c * (y + 0.044715 * y * y * y)))


def kernel(x, w_mat):
    x = x.astype(jnp.bfloat16)
    w = w_mat.astype(jnp.bfloat16)
    m, _ = x.shape

    def body(x_ref, w_ref, out_ref, send_buf, recv_buf, send_sems, recv_sems,
             credit_sem):
        my = lax.axis_index("i")
        left = lax.rem(my + N_DEV - 1, N_DEV)
        right = lax.rem(my + 1, N_DEV)

        barrier_sem = pltpu.get_barrier_semaphore()
        for nbr in (left, right):
            pl.semaphore_signal(
                barrier_sem, inc=1,
                device_id=(nbr,), device_id_type=pl.DeviceIdType.MESH,
            )
        pl.semaphore_wait(barrier_sem, 2)

        out_ref[...] = jnp.dot(
            x_ref[...], w_ref[...], preferred_element_type=jnp.float32
        )

        def hop(s, c_send, c_recv):
            send_buf[...] = out_ref[pl.ds(c_send * CHUNK, CHUNK), :].astype(
                jnp.bfloat16
            )
            if s >= 1:
                pl.semaphore_wait(credit_sem, 1)
            rdma = pltpu.make_async_remote_copy(
                src_ref=send_buf,
                dst_ref=recv_buf,
                send_sem=send_sems.at[s],
                recv_sem=recv_sems.at[s],
                device_id=(right,),
                device_id_type=pl.DeviceIdType.MESH,
            )
            rdma.start()
            rdma.wait()

        for s in range(N_DEV - 1):
            c_send = lax.rem(my - s + N_DEV, N_DEV)
            c_recv = lax.rem(my - s - 1 + N_DEV, N_DEV)
            hop(s, c_send, c_recv)
            acc = out_ref[pl.ds(c_recv * CHUNK, CHUNK), :] + recv_buf[
                ...
            ].astype(jnp.float32)
            out_ref[pl.ds(c_recv * CHUNK, CHUNK), :] = acc
            pl.semaphore_signal(
                credit_sem, inc=1,
                device_id=(left,), device_id_type=pl.DeviceIdType.MESH,
            )

        c_own = lax.rem(my + 1, N_DEV)
        out_ref[pl.ds(c_own * CHUNK, CHUNK), :] = _gelu(
            out_ref[pl.ds(c_own * CHUNK, CHUNK), :]
        )

        for t in range(N_DEV - 1):
            s = N_DEV - 1 + t
            c_send = lax.rem(my + 1 - t + N_DEV, N_DEV)
            c_recv = lax.rem(my - t + N_DEV, N_DEV)
            hop(s, c_send, c_recv)
            out_ref[pl.ds(c_recv * CHUNK, CHUNK), :] = recv_buf[...].astype(
                jnp.float32
            )
            if t < N_DEV - 2:
                pl.semaphore_signal(
                    credit_sem, inc=1,
                    device_id=(left,), device_id_type=pl.DeviceIdType.MESH,
                )

    return pl.pallas_call(
        body,
        out_shape=jax.ShapeDtypeStruct((m, N_OUT), jnp.float32),
        in_specs=[
            pl.BlockSpec(memory_space=pltpu.VMEM),
            pl.BlockSpec(memory_space=pltpu.VMEM),
        ],
        out_specs=pl.BlockSpec(memory_space=pltpu.VMEM),
        scratch_shapes=[
            pltpu.VMEM((CHUNK, N_OUT), jnp.bfloat16),
            pltpu.VMEM((CHUNK, N_OUT), jnp.bfloat16),
            pltpu.SemaphoreType.DMA((2 * (N_DEV - 1),)),
            pltpu.SemaphoreType.DMA((2 * (N_DEV - 1),)),
            pltpu.SemaphoreType.REGULAR,
        ],
        compiler_params=pltpu.CompilerParams(collective_id=0),
    )(x, w)


# baseline (device time: 366347 ns/iter reference)
import jax
import jax.numpy as jnp
from jax import lax
from jax.experimental import pallas as pl
from jax.experimental.pallas import tpu as pltpu

N_DEV = 4
M = 4096
N_OUT = 2048
CHUNK = M // N_DEV


def _gelu(y):
    c = 0.7978845608028654
    return 0.5 * y * (1.0 + jnp.tanh(c * (y + 0.044715 * y * y * y)))


def kernel(x, w_mat):
    x = x.astype(jnp.bfloat16)
    w = w_mat.astype(jnp.bfloat16)
    m, _ = x.shape

    def body(x_ref, w_ref, out_ref, send_buf, recv_buf, send_sems, recv_sems,
             credit_sem):
        my = lax.axis_index("i")
        left = lax.rem(my + N_DEV - 1, N_DEV)
        right = lax.rem(my + 1, N_DEV)

        barrier_sem = pltpu.get_barrier_semaphore()
        for nbr in (left, right):
            pl.semaphore_signal(
                barrier_sem, inc=1,
                device_id=(nbr,), device_id_type=pl.DeviceIdType.MESH,
            )
        pl.semaphore_wait(barrier_sem, 2)

        MM_BLK = 512
        for r in range(0, M, MM_BLK):
            out_ref[r:r + MM_BLK, :] = jnp.dot(
                x_ref[r:r + MM_BLK, :], w_ref[...],
                preferred_element_type=jnp.float32,
            )

        def hop(s, c_send, c_recv):
            send_buf[...] = out_ref[pl.ds(c_send * CHUNK, CHUNK), :].astype(
                jnp.bfloat16
            )
            if s >= 1:
                pl.semaphore_wait(credit_sem, 1)
            rdma = pltpu.make_async_remote_copy(
                src_ref=send_buf,
                dst_ref=recv_buf,
                send_sem=send_sems.at[s],
                recv_sem=recv_sems.at[s],
                device_id=(right,),
                device_id_type=pl.DeviceIdType.MESH,
            )
            rdma.start()
            rdma.wait()

        for s in range(N_DEV - 1):
            c_send = lax.rem(my - s + N_DEV, N_DEV)
            c_recv = lax.rem(my - s - 1 + N_DEV, N_DEV)
            hop(s, c_send, c_recv)
            acc = out_ref[pl.ds(c_recv * CHUNK, CHUNK), :] + recv_buf[
                ...
            ].astype(jnp.float32)
            out_ref[pl.ds(c_recv * CHUNK, CHUNK), :] = acc
            pl.semaphore_signal(
                credit_sem, inc=1,
                device_id=(left,), device_id_type=pl.DeviceIdType.MESH,
            )

        c_own = lax.rem(my + 1, N_DEV)
        out_ref[pl.ds(c_own * CHUNK, CHUNK), :] = _gelu(
            out_ref[pl.ds(c_own * CHUNK, CHUNK), :]
        )

        for t in range(N_DEV - 1):
            s = N_DEV - 1 + t
            c_send = lax.rem(my + 1 - t + N_DEV, N_DEV)
            c_recv = lax.rem(my - t + N_DEV, N_DEV)
            hop(s, c_send, c_recv)
            out_ref[pl.ds(c_recv * CHUNK, CHUNK), :] = recv_buf[...].astype(
                jnp.float32
            )
            if t < N_DEV - 2:
                pl.semaphore_signal(
                    credit_sem, inc=1,
                    device_id=(left,), device_id_type=pl.DeviceIdType.MESH,
                )

    return pl.pallas_call(
        body,
        out_shape=jax.ShapeDtypeStruct((m, N_OUT), jnp.float32),
        in_specs=[
            pl.BlockSpec(memory_space=pltpu.VMEM),
            pl.BlockSpec(memory_space=pltpu.VMEM),
        ],
        out_specs=pl.BlockSpec(memory_space=pltpu.VMEM),
        scratch_shapes=[
            pltpu.VMEM((CHUNK, N_OUT), jnp.bfloat16),
            pltpu.VMEM((CHUNK, N_OUT), jnp.bfloat16),
            pltpu.SemaphoreType.DMA((2 * (N_DEV - 1),)),
            pltpu.SemaphoreType.DMA((2 * (N_DEV - 1),)),
            pltpu.SemaphoreType.REGULAR,
        ],
        compiler_params=pltpu.CompilerParams(
            collective_id=0,
            vmem_limit_bytes=64 * 1024 * 1024,
        ),
    )(x, w)


# device time: 217710 ns/iter; 1.6827x vs baseline; 1.6827x over previous
import jax
import jax.numpy as jnp
from jax import lax
from jax.experimental import pallas as pl
from jax.experimental.pallas import tpu as pltpu

N_DEV = 4
M = 4096
N_OUT = 2048
CHUNK = M // N_DEV
HALF = N_OUT // 2
N_HOPS = 2 * (N_DEV - 1)


def _gelu(y):
    c = 0.7978845608028654
    return 0.5 * y * (1.0 + jnp.tanh(c * (y + 0.044715 * y * y * y)))


def kernel(x, w_mat):
    x = x.astype(jnp.bfloat16)
    w = w_mat.astype(jnp.bfloat16)
    m, _ = x.shape

    def body(x_ref, w_ref, out_ref,
             send_a, recv_a, send_b, recv_b,
             send_sems_a, recv_sems_a, send_sems_b, recv_sems_b,
             credit_a, credit_b):
        my = lax.axis_index("i")
        left = lax.rem(my + N_DEV - 1, N_DEV)
        right = lax.rem(my + 1, N_DEV)

        barrier_sem = pltpu.get_barrier_semaphore()
        for nbr in (left, right):
            pl.semaphore_signal(
                barrier_sem, inc=1,
                device_id=(nbr,), device_id_type=pl.DeviceIdType.MESH,
            )
        pl.semaphore_wait(barrier_sem, 2)

        def compute_chunk(c):
            for h in range(CHUNK // 512):
                r = c * CHUNK + h * 512
                out_ref[pl.ds(r, 512), :] = jnp.dot(
                    x_ref[pl.ds(r, 512), :], w_ref[...],
                    preferred_element_type=jnp.float32,
                )

        def fill_send(c_a, c_b):
            send_a[...] = out_ref[pl.ds(c_a * CHUNK, CHUNK), :HALF].astype(
                jnp.bfloat16)
            send_b[...] = out_ref[pl.ds(c_b * CHUNK, CHUNK), HALF:].astype(
                jnp.bfloat16)

        def make_rdmas(s):
            ra = pltpu.make_async_remote_copy(
                src_ref=send_a, dst_ref=recv_a,
                send_sem=send_sems_a.at[s], recv_sem=recv_sems_a.at[s],
                device_id=(right,), device_id_type=pl.DeviceIdType.MESH,
            )
            rb = pltpu.make_async_remote_copy(
                src_ref=send_b, dst_ref=recv_b,
                send_sem=send_sems_b.at[s], recv_sem=recv_sems_b.at[s],
                device_id=(left,), device_id_type=pl.DeviceIdType.MESH,
            )
            return ra, rb

        def accum(c_a, c_b):
            va = out_ref[pl.ds(c_a * CHUNK, CHUNK), :HALF] + recv_a[
                ...].astype(jnp.float32)
            out_ref[pl.ds(c_a * CHUNK, CHUNK), :HALF] = va
            vb = out_ref[pl.ds(c_b * CHUNK, CHUNK), HALF:] + recv_b[
                ...].astype(jnp.float32)
            out_ref[pl.ds(c_b * CHUNK, CHUNK), HALF:] = vb

        def credits():
            pl.semaphore_signal(
                credit_a, inc=1,
                device_id=(left,), device_id_type=pl.DeviceIdType.MESH,
            )
            pl.semaphore_signal(
                credit_b, inc=1,
                device_id=(right,), device_id_type=pl.DeviceIdType.MESH,
            )

        compute_chunk(my)
        fill_send(my, my)
        ra, rb = make_rdmas(0)
        ra.start()
        rb.start()
        for j in range(1, N_DEV):
            compute_chunk(lax.rem(my + j, N_DEV))
        ra.wait()
        rb.wait()
        accum(lax.rem(my - 1 + N_DEV, N_DEV), lax.rem(my + 1, N_DEV))
        credits()

        for s in range(1, N_DEV - 1):
            fill_send(lax.rem(my - s + N_DEV, N_DEV), lax.rem(my + s, N_DEV))
            pl.semaphore_wait(credit_a, 1)
            pl.semaphore_wait(credit_b, 1)
            ra, rb = make_rdmas(s)
            ra.start()
            rb.start()
            ra.wait()
            rb.wait()
            accum(lax.rem(my - s - 1 + N_DEV, N_DEV),
                  lax.rem(my + s + 1, N_DEV))
            credits()

        own_a = lax.rem(my + 1, N_DEV)
        own_b = lax.rem(my + 3, N_DEV)
        out_ref[pl.ds(own_a * CHUNK, CHUNK), :HALF] = _gelu(
            out_ref[pl.ds(own_a * CHUNK, CHUNK), :HALF])
        out_ref[pl.ds(own_b * CHUNK, CHUNK), HALF:] = _gelu(
            out_ref[pl.ds(own_b * CHUNK, CHUNK), HALF:])

        for t in range(N_DEV - 1):
            s = N_DEV - 1 + t
            fill_send(lax.rem(my + 1 - t + N_DEV, N_DEV),
                      lax.rem(my + 3 + t, N_DEV))
            pl.semaphore_wait(credit_a, 1)
            pl.semaphore_wait(credit_b, 1)
            ra, rb = make_rdmas(s)
            ra.start()
            rb.start()
            ra.wait()
            rb.wait()
            c_recv_a = lax.rem(my - t + N_DEV, N_DEV)
            c_recv_b = lax.rem(my + t, N_DEV)
            out_ref[pl.ds(c_recv_a * CHUNK, CHUNK), :HALF] = recv_a[
                ...].astype(jnp.float32)
            out_ref[pl.ds(c_recv_b * CHUNK, CHUNK), HALF:] = recv_b[
                ...].astype(jnp.float32)
            if t < N_DEV - 2:
                credits()

    return pl.pallas_call(
        body,
        out_shape=jax.ShapeDtypeStruct((m, N_OUT), jnp.float32),
        in_specs=[
            pl.BlockSpec(memory_space=pltpu.VMEM),
            pl.BlockSpec(memory_space=pltpu.VMEM),
        ],
        out_specs=pl.BlockSpec(memory_space=pltpu.VMEM),
        scratch_shapes=[
            pltpu.VMEM((CHUNK, HALF), jnp.bfloat16),
            pltpu.VMEM((CHUNK, HALF), jnp.bfloat16),
            pltpu.VMEM((CHUNK, HALF), jnp.bfloat16),
            pltpu.VMEM((CHUNK, HALF), jnp.bfloat16),
            pltpu.SemaphoreType.DMA((N_HOPS,)),
            pltpu.SemaphoreType.DMA((N_HOPS,)),
            pltpu.SemaphoreType.DMA((N_HOPS,)),
            pltpu.SemaphoreType.DMA((N_HOPS,)),
            pltpu.SemaphoreType.REGULAR,
            pltpu.SemaphoreType.REGULAR,
        ],
        compiler_params=pltpu.CompilerParams(
            collective_id=0,
            vmem_limit_bytes=64 * 1024 * 1024,
        ),
    )(x, w)


# device time: 200962 ns/iter; 1.8230x vs baseline; 1.0833x over previous
import jax
import jax.numpy as jnp
from jax import lax
from jax.experimental import pallas as pl
from jax.experimental.pallas import tpu as pltpu

N_DEV = 4
M = 4096
N_OUT = 2048
CHUNK = M // N_DEV
HALF = N_OUT // 2
SUB = 512
N_SEMS = 12


def _gelu(y):
    c = 0.7978845608028654
    return 0.5 * y * (1.0 + jnp.tanh(c * (y + 0.044715 * y * y * y)))


def kernel(x, w_mat):
    x = x.astype(jnp.bfloat16)
    w = w_mat.astype(jnp.bfloat16)
    m, _ = x.shape

    def body(x_ref, w_ref, out_ref,
             send_a, send_b, slots_a, slots_b,
             ssem_a, rsem_a, ssem_b, rsem_b,
             credit_a, credit_b):
        my = lax.axis_index("i")
        left = lax.rem(my + N_DEV - 1, N_DEV)
        right = lax.rem(my + 1, N_DEV)

        barrier_sem = pltpu.get_barrier_semaphore()
        for nbr in (left, right):
            pl.semaphore_signal(
                barrier_sem, inc=1,
                device_id=(nbr,), device_id_type=pl.DeviceIdType.MESH,
            )
        pl.semaphore_wait(barrier_sem, 2)

        def rdma_a(k, h, src_ref):
            return pltpu.make_async_remote_copy(
                src_ref=src_ref,
                dst_ref=slots_a.at[k % 3, pl.ds(h * SUB, SUB), :],
                send_sem=ssem_a.at[k * 2 + h], recv_sem=rsem_a.at[k * 2 + h],
                device_id=(right,), device_id_type=pl.DeviceIdType.MESH,
            )

        def rdma_b(k, h, src_ref):
            return pltpu.make_async_remote_copy(
                src_ref=src_ref,
                dst_ref=slots_b.at[k % 3, pl.ds(h * SUB, SUB), :],
                send_sem=ssem_b.at[k * 2 + h], recv_sem=rsem_b.at[k * 2 + h],
                device_id=(left,), device_id_type=pl.DeviceIdType.MESH,
            )

        rdmas_a = {}
        rdmas_b = {}

        for h in range(2):
            r = my * CHUNK + h * SUB
            blk = jnp.dot(x_ref[pl.ds(r, SUB), :], w_ref[...],
                          preferred_element_type=jnp.float32)
            out_ref[pl.ds(r, SUB), :] = blk
            send_a[pl.ds(h * SUB, SUB), :] = blk[:, :HALF].astype(jnp.bfloat16)
            send_b[pl.ds(h * SUB, SUB), :] = blk[:, HALF:].astype(jnp.bfloat16)
            ra = rdma_a(0, h, send_a.at[pl.ds(h * SUB, SUB), :])
            rb = rdma_b(0, h, send_b.at[pl.ds(h * SUB, SUB), :])
            ra.start()
            rb.start()
            rdmas_a[(0, h)] = ra
            rdmas_b[(0, h)] = rb

        for j in range(1, N_DEV):
            c = lax.rem(my + j, N_DEV)
            for h in range(2):
                r = c * CHUNK + h * SUB
                out_ref[pl.ds(r, SUB), :] = jnp.dot(
                    x_ref[pl.ds(r, SUB), :], w_ref[...],
                    preferred_element_type=jnp.float32,
                )

        for s in range(N_DEV - 1):
            c_a = lax.rem(my - s - 1 + N_DEV, N_DEV)
            c_b = lax.rem(my + s + 1, N_DEV)
            for h in range(2):
                sub_a = pl.ds(c_a * CHUNK + h * SUB, SUB)
                sub_b = pl.ds(c_b * CHUNK + h * SUB, SUB)
                rdmas_a[(s, h)].wait_recv()
                acc_a = out_ref[sub_a, :HALF] + slots_a[
                    s, pl.ds(h * SUB, SUB), :].astype(jnp.float32)
                out_ref[sub_a, :HALF] = acc_a
                if s < N_DEV - 2:
                    rdmas_a[(s, h)].wait_send()
                    send_a[pl.ds(h * SUB, SUB), :] = acc_a.astype(jnp.bfloat16)
                    ra = rdma_a(s + 1, h, send_a.at[pl.ds(h * SUB, SUB), :])
                    ra.start()
                    rdmas_a[(s + 1, h)] = ra
                rdmas_b[(s, h)].wait_recv()
                acc_b = out_ref[sub_b, HALF:] + slots_b[
                    s, pl.ds(h * SUB, SUB), :].astype(jnp.float32)
                out_ref[sub_b, HALF:] = acc_b
                if s < N_DEV - 2:
                    rdmas_b[(s, h)].wait_send()
                    send_b[pl.ds(h * SUB, SUB), :] = acc_b.astype(jnp.bfloat16)
                    rb = rdma_b(s + 1, h, send_b.at[pl.ds(h * SUB, SUB), :])
                    rb.start()
                    rdmas_b[(s + 1, h)] = rb

        pl.semaphore_signal(
            credit_a, inc=1,
            device_id=(left,), device_id_type=pl.DeviceIdType.MESH,
        )
        pl.semaphore_signal(
            credit_b, inc=1,
            device_id=(right,), device_id_type=pl.DeviceIdType.MESH,
        )

        own_a = lax.rem(my + 1, N_DEV)
        own_b = lax.rem(my + 3, N_DEV)
        for h in range(2):
            rdmas_a[(2, h)].wait_send()
            rdmas_b[(2, h)].wait_send()
        for h in range(2):
            sub_a = pl.ds(own_a * CHUNK + h * SUB, SUB)
            g_a = _gelu(out_ref[sub_a, :HALF])
            out_ref[sub_a, :HALF] = g_a
            send_a[pl.ds(h * SUB, SUB), :] = g_a.astype(jnp.bfloat16)
            sub_b = pl.ds(own_b * CHUNK + h * SUB, SUB)
            g_b = _gelu(out_ref[sub_b, HALF:])
            out_ref[sub_b, HALF:] = g_b
            send_b[pl.ds(h * SUB, SUB), :] = g_b.astype(jnp.bfloat16)

        pl.semaphore_wait(credit_a, 1)
        pl.semaphore_wait(credit_b, 1)

        for h in range(2):
            ra = rdma_a(3, h, send_a.at[pl.ds(h * SUB, SUB), :])
            rb = rdma_b(3, h, send_b.at[pl.ds(h * SUB, SUB), :])
            ra.start()
            rb.start()
            rdmas_a[(3, h)] = ra
            rdmas_b[(3, h)] = rb

        for t in range(1, N_DEV - 1):
            k = 3 + t
            c_a = lax.rem(my - t + 1 + N_DEV, N_DEV)
            c_b = lax.rem(my + t - 1, N_DEV)
            for h in range(2):
                hs = pl.ds(h * SUB, SUB)
                rdmas_a[(k - 1, h)].wait_recv()
                ra = rdma_a(k, h, slots_a.at[(k - 1) % 3, hs, :])
                ra.start()
                rdmas_a[(k, h)] = ra
                rdmas_b[(k - 1, h)].wait_recv()
                rb = rdma_b(k, h, slots_b.at[(k - 1) % 3, hs, :])
                rb.start()
                rdmas_b[(k, h)] = rb
                out_ref[pl.ds(c_a * CHUNK + h * SUB, SUB), :HALF] = slots_a[
                    (k - 1) % 3, hs, :].astype(jnp.float32)
                out_ref[pl.ds(c_b * CHUNK + h * SUB, SUB), HALF:] = slots_b[
                    (k - 1) % 3, hs, :].astype(jnp.float32)

        c_a = lax.rem(my + 2, N_DEV)
        c_b = lax.rem(my + 2, N_DEV)
        for h in range(2):
            hs = pl.ds(h * SUB, SUB)
            rdmas_a[(5, h)].wait_recv()
            out_ref[pl.ds(c_a * CHUNK + h * SUB, SUB), :HALF] = slots_a[
                2, hs, :].astype(jnp.float32)
            rdmas_b[(5, h)].wait_recv()
            out_ref[pl.ds(c_b * CHUNK + h * SUB, SUB), HALF:] = slots_b[
                2, hs, :].astype(jnp.float32)

        for k in range(3, 6):
            for h in range(2):
                rdmas_a[(k, h)].wait_send()
                rdmas_b[(k, h)].wait_send()

    return pl.pallas_call(
        body,
        out_shape=jax.ShapeDtypeStruct((m, N_OUT), jnp.float32),
        in_specs=[
            pl.BlockSpec(memory_space=pltpu.VMEM),
            pl.BlockSpec(memory_space=pltpu.VMEM),
        ],
        out_specs=pl.BlockSpec(memory_space=pltpu.VMEM),
        scratch_shapes=[
            pltpu.VMEM((CHUNK, HALF), jnp.bfloat16),
            pltpu.VMEM((CHUNK, HALF), jnp.bfloat16),
            pltpu.VMEM((3, CHUNK, HALF), jnp.bfloat16),
            pltpu.VMEM((3, CHUNK, HALF), jnp.bfloat16),
            pltpu.SemaphoreType.DMA((N_SEMS,)),
            pltpu.SemaphoreType.DMA((N_SEMS,)),
            pltpu.SemaphoreType.DMA((N_SEMS,)),
            pltpu.SemaphoreType.DMA((N_SEMS,)),
            pltpu.SemaphoreType.REGULAR,
            pltpu.SemaphoreType.REGULAR,
        ],
        compiler_params=pltpu.CompilerParams(
            collective_id=0,
            vmem_limit_bytes=64 * 1024 * 1024,
        ),
    )(x, w)


# device time: 191790 ns/iter; 1.9101x vs baseline; 1.0478x over previous
import jax
import jax.numpy as jnp
from jax import lax
from jax.experimental import pallas as pl
from jax.experimental.pallas import tpu as pltpu

N_DEV = 4
M = 4096
N_OUT = 2048
CHUNK = M // N_DEV
HALF = N_OUT // 2
SUB = 512
N_SEMS = 12


def _gelu(y):
    c = 0.7978845608028654
    return 0.5 * y * (1.0 + jnp.tanh(c * (y + 0.044715 * y * y * y)))


def kernel(x, w_mat):
    x = x.astype(jnp.bfloat16)
    w = w_mat.astype(jnp.bfloat16)
    m, _ = x.shape

    def body(x_ref, w_ref, out_ref,
             pA, pB, slots_a, slots_b, stage_a, stage_b,
             ssem_a, rsem_a, ssem_b, rsem_b, csem_a, csem_b,
             credit_a, credit_b):
        my = lax.axis_index("i")
        left = lax.rem(my + N_DEV - 1, N_DEV)
        right = lax.rem(my + 1, N_DEV)

        barrier_sem = pltpu.get_barrier_semaphore()
        for nbr in (left, right):
            pl.semaphore_signal(
                barrier_sem, inc=1,
                device_id=(nbr,), device_id_type=pl.DeviceIdType.MESH,
            )
        pl.semaphore_wait(barrier_sem, 2)

        def rdma_a(k, h, src_ref):
            return pltpu.make_async_remote_copy(
                src_ref=src_ref,
                dst_ref=slots_a.at[k % 3, pl.ds(h * SUB, SUB), :],
                send_sem=ssem_a.at[k * 2 + h], recv_sem=rsem_a.at[k * 2 + h],
                device_id=(right,), device_id_type=pl.DeviceIdType.MESH,
            )

        def rdma_b(k, h, src_ref):
            return pltpu.make_async_remote_copy(
                src_ref=src_ref,
                dst_ref=slots_b.at[k % 3, pl.ds(h * SUB, SUB), :],
                send_sem=ssem_b.at[k * 2 + h], recv_sem=rsem_b.at[k * 2 + h],
                device_id=(left,), device_id_type=pl.DeviceIdType.MESH,
            )

        rdmas_a = {}
        rdmas_b = {}
        copies_a = {}
        copies_b = {}

        def store_out(ring, p, c, h, value_f32):
            stage, csem, copies, col0 = (
                (stage_a, csem_a, copies_a, 0) if ring == 0
                else (stage_b, csem_b, copies_b, HALF)
            )
            if p in copies:
                copies[p].wait()
            stage[p, :, :] = value_f32
            cp = pltpu.make_async_copy(
                stage.at[p],
                out_ref.at[pl.ds(c * CHUNK + h * SUB, SUB),
                           pl.ds(col0, HALF)],
                csem.at[p],
            )
            cp.start()
            copies[p] = cp

        for h in range(2):
            r = my * CHUNK + h * SUB
            blk = jnp.dot(x_ref[pl.ds(r, SUB), :], w_ref[...],
                          preferred_element_type=jnp.float32)
            pA[my, pl.ds(h * SUB, SUB), :] = blk[:, :HALF].astype(jnp.bfloat16)
            pB[my, pl.ds(h * SUB, SUB), :] = blk[:, HALF:].astype(jnp.bfloat16)
            ra = rdma_a(0, h, pA.at[my, pl.ds(h * SUB, SUB), :])
            rb = rdma_b(0, h, pB.at[my, pl.ds(h * SUB, SUB), :])
            ra.start()
            rb.start()
            rdmas_a[(0, h)] = ra
            rdmas_b[(0, h)] = rb

        for j in (1, 3, 2):
            c = lax.rem(my + j, N_DEV)
            for h in range(2):
                r = c * CHUNK + h * SUB
                blk = jnp.dot(x_ref[pl.ds(r, SUB), :], w_ref[...],
                              preferred_element_type=jnp.float32)
                pA[c, pl.ds(h * SUB, SUB), :] = blk[:, :HALF].astype(
                    jnp.bfloat16)
                pB[c, pl.ds(h * SUB, SUB), :] = blk[:, HALF:].astype(
                    jnp.bfloat16)

        for s in range(N_DEV - 1):
            c_a = lax.rem(my - s - 1 + N_DEV, N_DEV)
            c_b = lax.rem(my + s + 1, N_DEV)
            for h in range(2):
                hs = pl.ds(h * SUB, SUB)
                rdmas_a[(s, h)].wait_recv()
                pA[c_a, hs, :] = pA[c_a, hs, :] + slots_a[s, hs, :]
                if s < N_DEV - 2:
                    ra = rdma_a(s + 1, h, pA.at[c_a, hs, :])
                    ra.start()
                    rdmas_a[(s + 1, h)] = ra
                rdmas_b[(s, h)].wait_recv()
                pB[c_b, hs, :] = pB[c_b, hs, :] + slots_b[s, hs, :]
                if s < N_DEV - 2:
                    rb = rdma_b(s + 1, h, pB.at[c_b, hs, :])
                    rb.start()
                    rdmas_b[(s + 1, h)] = rb

        pl.semaphore_signal(
            credit_a, inc=1,
            device_id=(left,), device_id_type=pl.DeviceIdType.MESH,
        )
        pl.semaphore_signal(
            credit_b, inc=1,
            device_id=(right,), device_id_type=pl.DeviceIdType.MESH,
        )

        own_a = lax.rem(my + 1, N_DEV)
        own_b = lax.rem(my + 3, N_DEV)
        for h in range(2):
            hs = pl.ds(h * SUB, SUB)
            g_a = _gelu(pA[own_a, hs, :].astype(jnp.float32))
            pA[own_a, hs, :] = g_a.astype(jnp.bfloat16)
            store_out(0, h, own_a, h, g_a)
            g_b = _gelu(pB[own_b, hs, :].astype(jnp.float32))
            pB[own_b, hs, :] = g_b.astype(jnp.bfloat16)
            store_out(1, h, own_b, h, g_b)

        pl.semaphore_wait(credit_a, 1)
        pl.semaphore_wait(credit_b, 1)

        for h in range(2):
            ra = rdma_a(3, h, pA.at[own_a, pl.ds(h * SUB, SUB), :])
            rb = rdma_b(3, h, pB.at[own_b, pl.ds(h * SUB, SUB), :])
            ra.start()
            rb.start()
            rdmas_a[(3, h)] = ra
            rdmas_b[(3, h)] = rb

        for t in range(1, N_DEV - 1):
            k = 3 + t
            c_a = lax.rem(my - t + 1 + N_DEV, N_DEV)
            c_b = lax.rem(my + t - 1, N_DEV)
            for h in range(2):
                hs = pl.ds(h * SUB, SUB)
                rdmas_a[(k - 1, h)].wait_recv()
                ra = rdma_a(k, h, slots_a.at[(k - 1) % 3, hs, :])
                ra.start()
                rdmas_a[(k, h)] = ra
                rdmas_b[(k - 1, h)].wait_recv()
                rb = rdma_b(k, h, slots_b.at[(k - 1) % 3, hs, :])
                rb.start()
                rdmas_b[(k, h)] = rb
                store_out(0, h, c_a, h,
                          slots_a[(k - 1) % 3, hs, :].astype(jnp.float32))
                store_out(1, h, c_b, h,
                          slots_b[(k - 1) % 3, hs, :].astype(jnp.float32))

        c_fin = lax.rem(my + 2, N_DEV)
        for h in range(2):
            hs = pl.ds(h * SUB, SUB)
            rdmas_a[(5, h)].wait_recv()
            store_out(0, h, c_fin, h, slots_a[2, hs, :].astype(jnp.float32))
            rdmas_b[(5, h)].wait_recv()
            store_out(1, h, c_fin, h, slots_b[2, hs, :].astype(jnp.float32))

        for kh, r in rdmas_a.items():
            r.wait_send()
        for kh, r in rdmas_b.items():
            r.wait_send()
        for cp in copies_a.values():
            cp.wait()
        for cp in copies_b.values():
            cp.wait()

    return pl.pallas_call(
        body,
        out_shape=jax.ShapeDtypeStruct((m, N_OUT), jnp.float32),
        in_specs=[
            pl.BlockSpec(memory_space=pltpu.VMEM),
            pl.BlockSpec(memory_space=pltpu.VMEM),
        ],
        out_specs=pl.BlockSpec(memory_space=pl.ANY),
        scratch_shapes=[
            pltpu.VMEM((N_DEV, CHUNK, HALF), jnp.bfloat16),
            pltpu.VMEM((N_DEV, CHUNK, HALF), jnp.bfloat16),
            pltpu.VMEM((3, CHUNK, HALF), jnp.bfloat16),
            pltpu.VMEM((3, CHUNK, HALF), jnp.bfloat16),
            pltpu.VMEM((2, SUB, HALF), jnp.float32),
            pltpu.VMEM((2, SUB, HALF), jnp.float32),
            pltpu.SemaphoreType.DMA((N_SEMS,)),
            pltpu.SemaphoreType.DMA((N_SEMS,)),
            pltpu.SemaphoreType.DMA((N_SEMS,)),
            pltpu.SemaphoreType.DMA((N_SEMS,)),
            pltpu.SemaphoreType.DMA((2,)),
            pltpu.SemaphoreType.DMA((2,)),
            pltpu.SemaphoreType.REGULAR,
            pltpu.SemaphoreType.REGULAR,
        ],
        compiler_params=pltpu.CompilerParams(
            collective_id=0,
            vmem_limit_bytes=64 * 1024 * 1024,
        ),
    )(x, w)


# device time: 187210 ns/iter; 1.9569x vs baseline; 1.0245x over previous
import jax
import jax.numpy as jnp
from jax import lax
from jax.experimental import pallas as pl
from jax.experimental.pallas import tpu as pltpu

N_DEV = 4
M = 4096
N_OUT = 2048
CHUNK = M // N_DEV
HALF = N_OUT // 2
SUB = 512
N_SEMS = 12


def _gelu(y):
    c = 0.7978845608028654
    return 0.5 * y * (1.0 + jnp.tanh(c * (y + 0.044715 * y * y * y)))


def kernel(x, w_mat):
    x = x.astype(jnp.bfloat16)
    w = w_mat.astype(jnp.bfloat16)
    m, _ = x.shape

    def body(x_ref, w_ref, out_ref,
             pA, pB, slots_a, slots_b, stage_a, stage_b,
             ssem_a, rsem_a, ssem_b, rsem_b, csem_a, csem_b,
             credit_a, credit_b):
        my = lax.axis_index("i")
        left = lax.rem(my + N_DEV - 1, N_DEV)
        right = lax.rem(my + 1, N_DEV)

        barrier_sem = pltpu.get_barrier_semaphore()
        for nbr in (left, right):
            pl.semaphore_signal(
                barrier_sem, inc=1,
                device_id=(nbr,), device_id_type=pl.DeviceIdType.MESH,
            )
        pl.semaphore_wait(barrier_sem, 2)

        def rdma_a(k, h, src_ref):
            return pltpu.make_async_remote_copy(
                src_ref=src_ref,
                dst_ref=slots_a.at[k % 3, pl.ds(h * SUB, SUB), :],
                send_sem=ssem_a.at[k * 2 + h], recv_sem=rsem_a.at[k * 2 + h],
                device_id=(right,), device_id_type=pl.DeviceIdType.MESH,
            )

        def rdma_b(k, h, src_ref):
            return pltpu.make_async_remote_copy(
                src_ref=src_ref,
                dst_ref=slots_b.at[k % 3, pl.ds(h * SUB, SUB), :],
                send_sem=ssem_b.at[k * 2 + h], recv_sem=rsem_b.at[k * 2 + h],
                device_id=(left,), device_id_type=pl.DeviceIdType.MESH,
            )

        rdmas_a = {}
        rdmas_b = {}
        copies_a = {}
        copies_b = {}

        def store_out(ring, p, c, h, value_f32):
            stage, csem, copies, col0 = (
                (stage_a, csem_a, copies_a, 0) if ring == 0
                else (stage_b, csem_b, copies_b, HALF)
            )
            if p in copies:
                copies[p].wait()
            stage[p, :, :] = value_f32
            cp = pltpu.make_async_copy(
                stage.at[p],
                out_ref.at[pl.ds(c * CHUNK + h * SUB, SUB),
                           pl.ds(col0, HALF)],
                csem.at[p],
            )
            cp.start()
            copies[p] = cp

        for h in range(2):
            r = my * CHUNK + h * SUB
            blk = jnp.dot(x_ref[pl.ds(r, SUB), :], w_ref[...],
                          preferred_element_type=jnp.float32)
            pA[my, pl.ds(h * SUB, SUB), :] = blk[:, :HALF].astype(jnp.bfloat16)
            pB[my, pl.ds(h * SUB, SUB), :] = blk[:, HALF:].astype(jnp.bfloat16)
            ra = rdma_a(0, h, pA.at[my, pl.ds(h * SUB, SUB), :])
            rb = rdma_b(0, h, pB.at[my, pl.ds(h * SUB, SUB), :])
            ra.start()
            rb.start()
            rdmas_a[(0, h)] = ra
            rdmas_b[(0, h)] = rb

        for j in (1, 3, 2):
            c = lax.rem(my + j, N_DEV)
            for h in range(2):
                r = c * CHUNK + h * SUB
                blk = jnp.dot(x_ref[pl.ds(r, SUB), :], w_ref[...],
                              preferred_element_type=jnp.float32)
                pA[c, pl.ds(h * SUB, SUB), :] = blk[:, :HALF].astype(
                    jnp.bfloat16)
                pB[c, pl.ds(h * SUB, SUB), :] = blk[:, HALF:].astype(
                    jnp.bfloat16)

        def send_credits():
            pl.semaphore_signal(
                credit_a, inc=1,
                device_id=(left,), device_id_type=pl.DeviceIdType.MESH,
            )
            pl.semaphore_signal(
                credit_b, inc=1,
                device_id=(right,), device_id_type=pl.DeviceIdType.MESH,
            )

        for s in range(N_DEV - 2):
            c_a = lax.rem(my - s - 1 + N_DEV, N_DEV)
            c_b = lax.rem(my + s + 1, N_DEV)
            for h in range(2):
                hs = pl.ds(h * SUB, SUB)
                rdmas_a[(s, h)].wait_recv()
                pA[c_a, hs, :] = pA[c_a, hs, :] + slots_a[s, hs, :]
                ra = rdma_a(s + 1, h, pA.at[c_a, hs, :])
                ra.start()
                rdmas_a[(s + 1, h)] = ra
                rdmas_b[(s, h)].wait_recv()
                pB[c_b, hs, :] = pB[c_b, hs, :] + slots_b[s, hs, :]
                rb = rdma_b(s + 1, h, pB.at[c_b, hs, :])
                rb.start()
                rdmas_b[(s + 1, h)] = rb
            send_credits()

        own_a = lax.rem(my + 1, N_DEV)
        own_b = lax.rem(my + 3, N_DEV)
        pl.semaphore_wait(credit_a, 1)
        pl.semaphore_wait(credit_b, 1)
        for h in range(2):
            hs = pl.ds(h * SUB, SUB)
            rdmas_a[(2, h)].wait_recv()
            g_a = _gelu(pA[own_a, hs, :].astype(jnp.float32)
                        + slots_a[2, hs, :].astype(jnp.float32))
            pA[own_a, hs, :] = g_a.astype(jnp.bfloat16)
            ra = rdma_a(3, h, pA.at[own_a, hs, :])
            ra.start()
            rdmas_a[(3, h)] = ra
            rdmas_b[(2, h)].wait_recv()
            g_b = _gelu(pB[own_b, hs, :].astype(jnp.float32)
                        + slots_b[2, hs, :].astype(jnp.float32))
            pB[own_b, hs, :] = g_b.astype(jnp.bfloat16)
            rb = rdma_b(3, h, pB.at[own_b, hs, :])
            rb.start()
            rdmas_b[(3, h)] = rb
            store_out(0, h, own_a, h, g_a)
            store_out(1, h, own_b, h, g_b)
        send_credits()

        for t in range(1, N_DEV - 1):
            k = 3 + t
            c_a = lax.rem(my - t + 1 + N_DEV, N_DEV)
            c_b = lax.rem(my + t - 1, N_DEV)
            pl.semaphore_wait(credit_a, 1)
            pl.semaphore_wait(credit_b, 1)
            for h in range(2):
                hs = pl.ds(h * SUB, SUB)
                rdmas_a[(k - 1, h)].wait_recv()
                ra = rdma_a(k, h, slots_a.at[(k - 1) % 3, hs, :])
                ra.start()
                rdmas_a[(k, h)] = ra
                rdmas_b[(k - 1, h)].wait_recv()
                rb = rdma_b(k, h, slots_b.at[(k - 1) % 3, hs, :])
                rb.start()
                rdmas_b[(k, h)] = rb
                store_out(0, h, c_a, h,
                          slots_a[(k - 1) % 3, hs, :].astype(jnp.float32))
                store_out(1, h, c_b, h,
                          slots_b[(k - 1) % 3, hs, :].astype(jnp.float32))

        c_fin = lax.rem(my + 2, N_DEV)
        for h in range(2):
            hs = pl.ds(h * SUB, SUB)
            rdmas_a[(5, h)].wait_recv()
            store_out(0, h, c_fin, h, slots_a[2, hs, :].astype(jnp.float32))
            rdmas_b[(5, h)].wait_recv()
            store_out(1, h, c_fin, h, slots_b[2, hs, :].astype(jnp.float32))

        for kh, r in rdmas_a.items():
            r.wait_send()
        for kh, r in rdmas_b.items():
            r.wait_send()
        for cp in copies_a.values():
            cp.wait()
        for cp in copies_b.values():
            cp.wait()

    return pl.pallas_call(
        body,
        out_shape=jax.ShapeDtypeStruct((m, N_OUT), jnp.float32),
        in_specs=[
            pl.BlockSpec(memory_space=pltpu.VMEM),
            pl.BlockSpec(memory_space=pltpu.VMEM),
        ],
        out_specs=pl.BlockSpec(memory_space=pl.ANY),
        scratch_shapes=[
            pltpu.VMEM((N_DEV, CHUNK, HALF), jnp.bfloat16),
            pltpu.VMEM((N_DEV, CHUNK, HALF), jnp.bfloat16),
            pltpu.VMEM((3, CHUNK, HALF), jnp.bfloat16),
            pltpu.VMEM((3, CHUNK, HALF), jnp.bfloat16),
            pltpu.VMEM((2, SUB, HALF), jnp.float32),
            pltpu.VMEM((2, SUB, HALF), jnp.float32),
            pltpu.SemaphoreType.DMA((N_SEMS,)),
            pltpu.SemaphoreType.DMA((N_SEMS,)),
            pltpu.SemaphoreType.DMA((N_SEMS,)),
            pltpu.SemaphoreType.DMA((N_SEMS,)),
            pltpu.SemaphoreType.DMA((2,)),
            pltpu.SemaphoreType.DMA((2,)),
            pltpu.SemaphoreType.REGULAR,
            pltpu.SemaphoreType.REGULAR,
        ],
        compiler_params=pltpu.CompilerParams(
            collective_id=0,
            vmem_limit_bytes=64 * 1024 * 1024,
        ),
    )(x, w)


# device time: 185598 ns/iter; 1.9739x vs baseline; 1.0087x over previous
import jax
import jax.numpy as jnp
from jax import lax
from jax.experimental import pallas as pl
from jax.experimental.pallas import tpu as pltpu

N_DEV = 4
M = 4096
N_OUT = 2048
CHUNK = M // N_DEV
HALF = N_OUT // 2
SUB = 256
B = CHUNK // SUB
N_SEMS = 6 * B


def _gelu(y):
    c = 0.7978845608028654
    return 0.5 * y * (1.0 + jnp.tanh(c * (y + 0.044715 * y * y * y)))


def kernel(x, w_mat):
    x = x.astype(jnp.bfloat16)
    w = w_mat.astype(jnp.bfloat16)
    m, _ = x.shape

    def body(x_ref, w_ref, out_ref,
             pA, pB, slots_a, slots_b, stage_a, stage_b,
             ssem_a, rsem_a, ssem_b, rsem_b, csem_a, csem_b,
             credit_a, credit_b):
        my = lax.axis_index("i")
        left = lax.rem(my + N_DEV - 1, N_DEV)
        right = lax.rem(my + 1, N_DEV)

        barrier_sem = pltpu.get_barrier_semaphore()
        for nbr in (left, right):
            pl.semaphore_signal(
                barrier_sem, inc=1,
                device_id=(nbr,), device_id_type=pl.DeviceIdType.MESH,
            )
        pl.semaphore_wait(barrier_sem, 2)

        def rdma_a(k, h, src_ref):
            return pltpu.make_async_remote_copy(
                src_ref=src_ref,
                dst_ref=slots_a.at[k % 3, pl.ds(h * SUB, SUB), :],
                send_sem=ssem_a.at[k * B + h], recv_sem=rsem_a.at[k * B + h],
                device_id=(right,), device_id_type=pl.DeviceIdType.MESH,
            )

        def rdma_b(k, h, src_ref):
            return pltpu.make_async_remote_copy(
                src_ref=src_ref,
                dst_ref=slots_b.at[k % 3, pl.ds(h * SUB, SUB), :],
                send_sem=ssem_b.at[k * B + h], recv_sem=rsem_b.at[k * B + h],
                device_id=(left,), device_id_type=pl.DeviceIdType.MESH,
            )

        rdmas_a = {}
        rdmas_b = {}
        copies_a = {}
        copies_b = {}

        def store_out(ring, p, c, h, value_f32):
            stage, csem, copies, col0 = (
                (stage_a, csem_a, copies_a, 0) if ring == 0
                else (stage_b, csem_b, copies_b, HALF)
            )
            if p in copies:
                copies[p].wait()
            stage[p, :, :] = value_f32
            cp = pltpu.make_async_copy(
                stage.at[p],
                out_ref.at[pl.ds(c * CHUNK + h * SUB, SUB),
                           pl.ds(col0, HALF)],
                csem.at[p],
            )
            cp.start()
            copies[p] = cp

        for h in range(B):
            r = my * CHUNK + h * SUB
            blk = jnp.dot(x_ref[pl.ds(r, SUB), :], w_ref[...],
                          preferred_element_type=jnp.float32)
            pA[my, pl.ds(h * SUB, SUB), :] = blk[:, :HALF].astype(jnp.bfloat16)
            pB[my, pl.ds(h * SUB, SUB), :] = blk[:, HALF:].astype(jnp.bfloat16)
            ra = rdma_a(0, h, pA.at[my, pl.ds(h * SUB, SUB), :])
            rb = rdma_b(0, h, pB.at[my, pl.ds(h * SUB, SUB), :])
            ra.start()
            rb.start()
            rdmas_a[(0, h)] = ra
            rdmas_b[(0, h)] = rb

        for j in (1, 3, 2):
            c = lax.rem(my + j, N_DEV)
            for h in range(2):
                r = c * CHUNK + h * 512
                blk = jnp.dot(x_ref[pl.ds(r, 512), :], w_ref[...],
                              preferred_element_type=jnp.float32)
                pA[c, pl.ds(h * 512, 512), :] = blk[:, :HALF].astype(
                    jnp.bfloat16)
                pB[c, pl.ds(h * 512, 512), :] = blk[:, HALF:].astype(
                    jnp.bfloat16)

        def send_credits():
            pl.semaphore_signal(
                credit_a, inc=1,
                device_id=(left,), device_id_type=pl.DeviceIdType.MESH,
            )
            pl.semaphore_signal(
                credit_b, inc=1,
                device_id=(right,), device_id_type=pl.DeviceIdType.MESH,
            )

        for s in range(N_DEV - 2):
            c_a = lax.rem(my - s - 1 + N_DEV, N_DEV)
            c_b = lax.rem(my + s + 1, N_DEV)
            for h in range(B):
                hs = pl.ds(h * SUB, SUB)
                rdmas_a[(s, h)].wait_recv()
                pA[c_a, hs, :] = pA[c_a, hs, :] + slots_a[s, hs, :]
                ra = rdma_a(s + 1, h, pA.at[c_a, hs, :])
                ra.start()
                rdmas_a[(s + 1, h)] = ra
                rdmas_b[(s, h)].wait_recv()
                pB[c_b, hs, :] = pB[c_b, hs, :] + slots_b[s, hs, :]
                rb = rdma_b(s + 1, h, pB.at[c_b, hs, :])
                rb.start()
                rdmas_b[(s + 1, h)] = rb
            send_credits()

        own_a = lax.rem(my + 1, N_DEV)
        own_b = lax.rem(my + 3, N_DEV)
        pl.semaphore_wait(credit_a, 1)
        pl.semaphore_wait(credit_b, 1)
        for h in range(B):
            hs = pl.ds(h * SUB, SUB)
            rdmas_a[(2, h)].wait_recv()
            g_a = _gelu(pA[own_a, hs, :].astype(jnp.float32)
                        + slots_a[2, hs, :].astype(jnp.float32))
            pA[own_a, hs, :] = g_a.astype(jnp.bfloat16)
            ra = rdma_a(3, h, pA.at[own_a, hs, :])
            ra.start()
            rdmas_a[(3, h)] = ra
            rdmas_b[(2, h)].wait_recv()
            g_b = _gelu(pB[own_b, hs, :].astype(jnp.float32)
                        + slots_b[2, hs, :].astype(jnp.float32))
            pB[own_b, hs, :] = g_b.astype(jnp.bfloat16)
            rb = rdma_b(3, h, pB.at[own_b, hs, :])
            rb.start()
            rdmas_b[(3, h)] = rb
            store_out(0, h % 2, own_a, h, g_a)
            store_out(1, h % 2, own_b, h, g_b)
        send_credits()

        for t in range(1, N_DEV - 1):
            k = 3 + t
            c_a = lax.rem(my - t + 1 + N_DEV, N_DEV)
            c_b = lax.rem(my + t - 1, N_DEV)
            pl.semaphore_wait(credit_a, 1)
            pl.semaphore_wait(credit_b, 1)
            for h in range(B):
                hs = pl.ds(h * SUB, SUB)
                rdmas_a[(k - 1, h)].wait_recv()
                ra = rdma_a(k, h, slots_a.at[(k - 1) % 3, hs, :])
                ra.start()
                rdmas_a[(k, h)] = ra
                rdmas_b[(k - 1, h)].wait_recv()
                rb = rdma_b(k, h, slots_b.at[(k - 1) % 3, hs, :])
                rb.start()
                rdmas_b[(k, h)] = rb
                store_out(0, h % 2, c_a, h,
                          slots_a[(k - 1) % 3, hs, :].astype(jnp.float32))
                store_out(1, h % 2, c_b, h,
                          slots_b[(k - 1) % 3, hs, :].astype(jnp.float32))

        c_fin = lax.rem(my + 2, N_DEV)
        for h in range(B):
            hs = pl.ds(h * SUB, SUB)
            rdmas_a[(5, h)].wait_recv()
            store_out(0, h % 2, c_fin, h, slots_a[2, hs, :].astype(jnp.float32))
            rdmas_b[(5, h)].wait_recv()
            store_out(1, h % 2, c_fin, h, slots_b[2, hs, :].astype(jnp.float32))

        for kh, r in rdmas_a.items():
            r.wait_send()
        for kh, r in rdmas_b.items():
            r.wait_send()
        for cp in copies_a.values():
            cp.wait()
        for cp in copies_b.values():
            cp.wait()

    return pl.pallas_call(
        body,
        out_shape=jax.ShapeDtypeStruct((m, N_OUT), jnp.float32),
        in_specs=[
            pl.BlockSpec(memory_space=pltpu.VMEM),
            pl.BlockSpec(memory_space=pltpu.VMEM),
        ],
        out_specs=pl.BlockSpec(memory_space=pl.ANY),
        scratch_shapes=[
            pltpu.VMEM((N_DEV, CHUNK, HALF), jnp.bfloat16),
            pltpu.VMEM((N_DEV, CHUNK, HALF), jnp.bfloat16),
            pltpu.VMEM((3, CHUNK, HALF), jnp.bfloat16),
            pltpu.VMEM((3, CHUNK, HALF), jnp.bfloat16),
            pltpu.VMEM((2, SUB, HALF), jnp.float32),
            pltpu.VMEM((2, SUB, HALF), jnp.float32),
            pltpu.SemaphoreType.DMA((N_SEMS,)),
            pltpu.SemaphoreType.DMA((N_SEMS,)),
            pltpu.SemaphoreType.DMA((N_SEMS,)),
            pltpu.SemaphoreType.DMA((N_SEMS,)),
            pltpu.SemaphoreType.DMA((2,)),
            pltpu.SemaphoreType.DMA((2,)),
            pltpu.SemaphoreType.REGULAR,
            pltpu.SemaphoreType.REGULAR,
        ],
        compiler_params=pltpu.CompilerParams(
            collective_id=0,
            vmem_limit_bytes=64 * 1024 * 1024,
        ),
    )(x, w)


# device time: 169399 ns/iter; 2.1626x vs baseline; 1.0956x over previous
import jax
import jax.numpy as jnp
from jax import lax
from jax.experimental import pallas as pl
from jax.experimental.pallas import tpu as pltpu

N_DEV = 4
M = 4096
N_OUT = 2048
CHUNK = M // N_DEV
HALF = N_OUT // 2
SUB = 256
B = CHUNK // SUB
N_SEMS = 6 * B


def _gelu(y):
    c = 0.7978845608028654
    return 0.5 * y * (1.0 + jnp.tanh(c * (y + 0.044715 * y * y * y)))


def kernel(x, w_mat):
    m, _ = x.shape

    def body(x_ref, w_ref, out_ref,
             pA, pB, slots_a, slots_b, stage_a, stage_b,
             w_bf, wstage, xstage,
             ssem_a, rsem_a, ssem_b, rsem_b, csem_a, csem_b,
             wsem, xsem,
             credit_a, credit_b):
        my = lax.axis_index("i")
        left = lax.rem(my + N_DEV - 1, N_DEV)
        right = lax.rem(my + 1, N_DEV)

        w_cp = pltpu.make_async_copy(w_ref, wstage, wsem)
        w_cp.start()

        x_dmas = {}

        def x_dma(i, c, h):
            cp = pltpu.make_async_copy(
                x_ref.at[pl.ds(c * CHUNK + h * SUB, SUB), :],
                xstage.at[i % 4],
                xsem.at[i % 4],
            )
            cp.start()
            x_dmas[i] = cp

        seq = [(j, h) for j in (0, 1, 3, 2) for h in range(B)]
        for i in range(4):
            j, h = seq[i]
            x_dma(i, lax.rem(my + j, N_DEV), h)

        barrier_sem = pltpu.get_barrier_semaphore()
        for nbr in (left, right):
            pl.semaphore_signal(
                barrier_sem, inc=1,
                device_id=(nbr,), device_id_type=pl.DeviceIdType.MESH,
            )
        pl.semaphore_wait(barrier_sem, 2)

        def rdma_a(k, h, src_ref):
            return pltpu.make_async_remote_copy(
                src_ref=src_ref,
                dst_ref=slots_a.at[k % 3, pl.ds(h * SUB, SUB), :],
                send_sem=ssem_a.at[k * B + h], recv_sem=rsem_a.at[k * B + h],
                device_id=(right,), device_id_type=pl.DeviceIdType.MESH,
            )

        def rdma_b(k, h, src_ref):
            return pltpu.make_async_remote_copy(
                src_ref=src_ref,
                dst_ref=slots_b.at[k % 3, pl.ds(h * SUB, SUB), :],
                send_sem=ssem_b.at[k * B + h], recv_sem=rsem_b.at[k * B + h],
                device_id=(left,), device_id_type=pl.DeviceIdType.MESH,
            )

        rdmas_a = {}
        rdmas_b = {}
        copies_a = {}
        copies_b = {}

        def store_out(ring, p, c, h, value_f32):
            stage, csem, copies, col0 = (
                (stage_a, csem_a, copies_a, 0) if ring == 0
                else (stage_b, csem_b, copies_b, HALF)
            )
            if p in copies:
                copies[p].wait()
            stage[p, :, :] = value_f32
            cp = pltpu.make_async_copy(
                stage.at[p],
                out_ref.at[pl.ds(c * CHUNK + h * SUB, SUB),
                           pl.ds(col0, HALF)],
                csem.at[p],
            )
            cp.start()
            copies[p] = cp

        w_cp.wait()
        w_bf[...] = wstage[...].astype(jnp.bfloat16)

        for i, (j, h) in enumerate(seq):
            c = lax.rem(my + j, N_DEV)
            hs = pl.ds(h * SUB, SUB)
            x_dmas[i].wait()
            blk = jnp.dot(xstage[i % 4].astype(jnp.bfloat16), w_bf[...],
                          preferred_element_type=jnp.float32)
            if i + 4 < len(seq):
                j2, h2 = seq[i + 4]
                x_dma(i + 4, lax.rem(my + j2, N_DEV), h2)
            pA[c, hs, :] = blk[:, :HALF].astype(jnp.bfloat16)
            pB[c, hs, :] = blk[:, HALF:].astype(jnp.bfloat16)
            if j == 0:
                ra = rdma_a(0, h, pA.at[my, hs, :])
                rb = rdma_b(0, h, pB.at[my, hs, :])
                ra.start()
                rb.start()
                rdmas_a[(0, h)] = ra
                rdmas_b[(0, h)] = rb

        def send_credits():
            pl.semaphore_signal(
                credit_a, inc=1,
                device_id=(left,), device_id_type=pl.DeviceIdType.MESH,
            )
            pl.semaphore_signal(
                credit_b, inc=1,
                device_id=(right,), device_id_type=pl.DeviceIdType.MESH,
            )

        for s in range(N_DEV - 2):
            c_a = lax.rem(my - s - 1 + N_DEV, N_DEV)
            c_b = lax.rem(my + s + 1, N_DEV)
            for h in range(B):
                hs = pl.ds(h * SUB, SUB)
                rdmas_a[(s, h)].wait_recv()
                pA[c_a, hs, :] = pA[c_a, hs, :] + slots_a[s, hs, :]
                ra = rdma_a(s + 1, h, pA.at[c_a, hs, :])
                ra.start()
                rdmas_a[(s + 1, h)] = ra
                rdmas_b[(s, h)].wait_recv()
                pB[c_b, hs, :] = pB[c_b, hs, :] + slots_b[s, hs, :]
                rb = rdma_b(s + 1, h, pB.at[c_b, hs, :])
                rb.start()
                rdmas_b[(s + 1, h)] = rb
            send_credits()

        own_a = lax.rem(my + 1, N_DEV)
        own_b = lax.rem(my + 3, N_DEV)
        pl.semaphore_wait(credit_a, 1)
        pl.semaphore_wait(credit_b, 1)
        for h in range(B):
            hs = pl.ds(h * SUB, SUB)
            rdmas_a[(2, h)].wait_recv()
            g_a = _gelu(pA[own_a, hs, :].astype(jnp.float32)
                        + slots_a[2, hs, :].astype(jnp.float32))
            pA[own_a, hs, :] = g_a.astype(jnp.bfloat16)
            ra = rdma_a(3, h, pA.at[own_a, hs, :])
            ra.start()
            rdmas_a[(3, h)] = ra
            rdmas_b[(2, h)].wait_recv()
            g_b = _gelu(pB[own_b, hs, :].astype(jnp.float32)
                        + slots_b[2, hs, :].astype(jnp.float32))
            pB[own_b, hs, :] = g_b.astype(jnp.bfloat16)
            rb = rdma_b(3, h, pB.at[own_b, hs, :])
            rb.start()
            rdmas_b[(3, h)] = rb
            store_out(0, h % 2, own_a, h, g_a)
            store_out(1, h % 2, own_b, h, g_b)
        send_credits()

        for t in range(1, N_DEV - 1):
            k = 3 + t
            c_a = lax.rem(my - t + 1 + N_DEV, N_DEV)
            c_b = lax.rem(my + t - 1, N_DEV)
            pl.semaphore_wait(credit_a, 1)
            pl.semaphore_wait(credit_b, 1)
            for h in range(B):
                hs = pl.ds(h * SUB, SUB)
                rdmas_a[(k - 1, h)].wait_recv()
                ra = rdma_a(k, h, slots_a.at[(k - 1) % 3, hs, :])
                ra.start()
                rdmas_a[(k, h)] = ra
                rdmas_b[(k - 1, h)].wait_recv()
                rb = rdma_b(k, h, slots_b.at[(k - 1) % 3, hs, :])
                rb.start()
                rdmas_b[(k, h)] = rb
                store_out(0, h % 2, c_a, h,
                          slots_a[(k - 1) % 3, hs, :].astype(jnp.float32))
                store_out(1, h % 2, c_b, h,
                          slots_b[(k - 1) % 3, hs, :].astype(jnp.float32))

        c_fin = lax.rem(my + 2, N_DEV)
        for h in range(B):
            hs = pl.ds(h * SUB, SUB)
            rdmas_a[(5, h)].wait_recv()
            store_out(0, h % 2, c_fin, h, slots_a[2, hs, :].astype(jnp.float32))
            rdmas_b[(5, h)].wait_recv()
            store_out(1, h % 2, c_fin, h, slots_b[2, hs, :].astype(jnp.float32))

        for kh, r in rdmas_a.items():
            r.wait_send()
        for kh, r in rdmas_b.items():
            r.wait_send()
        for cp in copies_a.values():
            cp.wait()
        for cp in copies_b.values():
            cp.wait()

    return pl.pallas_call(
        body,
        out_shape=jax.ShapeDtypeStruct((m, N_OUT), jnp.float32),
        in_specs=[
            pl.BlockSpec(memory_space=pl.ANY),
            pl.BlockSpec(memory_space=pl.ANY),
        ],
        out_specs=pl.BlockSpec(memory_space=pl.ANY),
        scratch_shapes=[
            pltpu.VMEM((N_DEV, CHUNK, HALF), jnp.bfloat16),
            pltpu.VMEM((N_DEV, CHUNK, HALF), jnp.bfloat16),
            pltpu.VMEM((3, CHUNK, HALF), jnp.bfloat16),
            pltpu.VMEM((3, CHUNK, HALF), jnp.bfloat16),
            pltpu.VMEM((2, SUB, HALF), jnp.float32),
            pltpu.VMEM((2, SUB, HALF), jnp.float32),
            pltpu.VMEM((1024, N_OUT), jnp.bfloat16),
            pltpu.VMEM((1024, N_OUT), jnp.float32),
            pltpu.VMEM((4, SUB, 1024), jnp.float32),
            pltpu.SemaphoreType.DMA((N_SEMS,)),
            pltpu.SemaphoreType.DMA((N_SEMS,)),
            pltpu.SemaphoreType.DMA((N_SEMS,)),
            pltpu.SemaphoreType.DMA((N_SEMS,)),
            pltpu.SemaphoreType.DMA((2,)),
            pltpu.SemaphoreType.DMA((2,)),
            pltpu.SemaphoreType.DMA,
            pltpu.SemaphoreType.DMA((4,)),
            pltpu.SemaphoreType.REGULAR,
            pltpu.SemaphoreType.REGULAR,
        ],
        compiler_params=pltpu.CompilerParams(
            collective_id=0,
            vmem_limit_bytes=64 * 1024 * 1024,
        ),
    )(x, w_mat)
